# baseline (speedup 1.0000x reference)
import jax
import jax.numpy as jnp
from jax import lax
from jax.experimental import pallas as pl

_TILE_B = 64


def _dot(a, b):
    return jnp.dot(a, b, preferred_element_type=jnp.float32)


def _body(idx8_ref, seq_ref, pho_ref, isl_ref,
          t_wday, t_hour, t_min, t_uid, t_did, t_gen, t_age, t_pro,
          t_vid, t_aid, t_c2, t_c1, t_up,
          uW1_ref, ub1_ref, uW2_ref, ub2_ref, uW3_ref, ub3_ref,
          pW1_ref, pb1_ref, pW2_ref, pb2_ref, pW3_ref, pb3_ref,
          out_ref):
    B = idx8_ref.shape[0]
    TB = _TILE_B
    NB = B // TB
    L = seq_ref.shape[2]
    NR = pho_ref.shape[0] // B
    PR = TB * NR
    VP = t_wday.shape[0]

    scalar_tabs = [t_wday, t_hour, t_min, t_uid, t_did, t_gen, t_age, t_pro]
    seq_tabs = [t_vid, t_aid, t_c2, t_c1, t_up]
    photo_tabs = [t_vid, t_aid, t_c2, t_c1, t_up, t_wday, t_hour, t_min]

    uW1 = uW1_ref[...]
    uW2 = uW2_ref[...]
    uW3 = uW3_ref[...]
    pW1 = pW1_ref[...]
    pW2 = pW2_ref[...]
    pW3 = pW3_ref[...]
    ub1 = ub1_ref[...]
    ub2 = ub2_ref[...]
    ub3 = ub3_ref[...]
    pb1 = pb1_ref[...]
    pb2 = pb2_ref[...]
    pb3 = pb3_ref[...]

    lane2 = lax.broadcasted_iota(jnp.int32, (TB, VP), 1)
    lane3 = lax.broadcasted_iota(jnp.int32, (TB, L, VP), 2)
    lanep = lax.broadcasted_iota(jnp.int32, (PR, VP), 1)

    def tile(i, _):
        idx8 = idx8_ref[pl.ds(i * TB, TB), :]
        seq = seq_ref[pl.ds(i * TB, TB), :, :]
        pho = pho_ref[pl.ds(i * PR, PR), :]
        isl = isl_ref[pl.ds(i * TB, TB), :]

        embs = []
        for f in range(8):
            oh = (idx8[:, f:f + 1] == lane2).astype(jnp.bfloat16)
            embs.append(_dot(oh, scalar_tabs[f][...]))
        for f in range(5):
            oh3 = (seq[:, f, :][:, :, None] == lane3).astype(jnp.float32)
            counts = jnp.sum(oh3, axis=1)
            embs.append(_dot(counts.astype(jnp.bfloat16),
                             seq_tabs[f][...]) * isl)
        u_in = jnp.concatenate(embs, axis=1)

        h = jnp.maximum(_dot(u_in, uW1) + ub1, 0.0)
        h = jnp.maximum(_dot(h, uW2) + ub2, 0.0)
        u_out = _dot(h, uW3) + ub3

        pembs = []
        for f in range(8):
            oh = (pho[:, f:f + 1] == lanep).astype(jnp.bfloat16)
            pembs.append(_dot(oh, photo_tabs[f][...]))
        p_in = jnp.concatenate(pembs, axis=1)

        h = jnp.maximum(_dot(p_in, pW1) + pb1, 0.0)
        h = jnp.maximum(_dot(h, pW2) + pb2, 0.0)
        p_out = _dot(h, pW3) + pb3

        p3 = p_out.reshape(TB, NR, p_out.shape[1])
        out_ref[pl.ds(i * TB, TB), :] = jnp.sum(p3 * u_out[:, None, :],
                                                axis=2)
        return 0

    lax.fori_loop(0, NB, tile, 0)


def kernel(request_wday, request_hour, request_min, uid, did, gender, age,
           province, seq_arr, seq_len, rank_pos_photos,
           uid_tab, did_tab, gender_tab, age_tab, province_tab, vid_tab,
           aid_tab, cate_two_tab, cate_one_tab, up_type_tab, wday_tab,
           hour_tab, min_tab,
           uW1, ub1, uW2, ub2, uW3, ub3, pW1, pb1, pW2, pb2, pW3, pb3):
    B, L, _ = seq_arr.shape
    NR = rank_pos_photos.shape[1]
    D = uid_tab.shape[1]
    V = uid_tab.shape[0]
    VP = 128

    idx8 = jnp.stack([request_wday, request_hour, request_min, uid, did,
                      gender, age, province], axis=1).astype(jnp.int32)
    seq_t = jnp.transpose(seq_arr.astype(jnp.int32), (0, 2, 1))
    photos = rank_pos_photos.astype(jnp.int32).reshape(B * NR, 8)
    inv_sl = (1.0 / seq_len.astype(jnp.float32)).reshape(B, 1)

    def pad(t):
        return jnp.zeros((VP, D), jnp.bfloat16).at[:V].set(
            t.astype(jnp.bfloat16))

    tabs = [pad(t) for t in (wday_tab, hour_tab, min_tab, uid_tab, did_tab,
                             gender_tab, age_tab, province_tab,
                             vid_tab, aid_tab, cate_two_tab, cate_one_tab,
                             up_type_tab)]

    weights = [uW1, ub1.reshape(1, -1), uW2, ub2.reshape(1, -1),
               uW3, ub3.reshape(1, -1), pW1, pb1.reshape(1, -1),
               pW2, pb2.reshape(1, -1), pW3, pb3.reshape(1, -1)]

    out = pl.pallas_call(
        _body,
        out_shape=jax.ShapeDtypeStruct((B, NR), jnp.float32),
    )(idx8, seq_t, photos, inv_sl, *tabs, *weights)
    return out
